# baseline (speedup 1.0000x reference)
import functools

import jax
import jax.numpy as jnp
from jax import lax
from jax.experimental import pallas as pl
from jax.experimental.pallas import tpu as pltpu
from jax.experimental.pallas import tpu_sc as plsc

_NUM_CLASSES = 1000
_EMBED = 128
_BATCH = 16384

_NC = 2
_NS = 16
_NW = _NC * _NS
_BPW = _BATCH // _NW
_R = 128
_K = _BPW // _R
_LANES = 16
_VPR = _EMBED // _LANES


@functools.partial(
    pl.kernel,
    out_type=jax.ShapeDtypeStruct((_BATCH, _EMBED), jnp.float32),
    mesh=plsc.VectorSubcoreMesh(core_axis_name="c", subcore_axis_name="s"),
    scratch_types=[
        pltpu.VMEM((1,), jnp.int32),
        pltpu.VMEM((_R, _EMBED), jnp.float32),
        pltpu.SemaphoreType.DMA,
        pltpu.SemaphoreType.DMA,
    ],
)
def _bcast_row(idx_hbm, table_hbm, out_hbm, idx_v, st_v, gsem, osem):
    wid = lax.axis_index("s") * _NC + lax.axis_index("c")
    base = wid * _BPW
    pltpu.sync_copy(idx_hbm, idx_v)
    pltpu.async_copy(table_hbm.at[idx_v], st_v.at[pl.ds(0, 1)], gsem).wait()
    regs = [st_v[0, pl.ds(c * _LANES, _LANES)] for c in range(_VPR)]
    half = _R // 2
    for r in range(1, half):
        for c in range(_VPR):
            st_v[r, pl.ds(c * _LANES, _LANES)] = regs[c]
    first = [
        pltpu.async_copy(
            st_v.at[pl.ds(0, half)],
            out_hbm.at[pl.ds(base + j * half, half)],
            osem,
        )
        for j in range(2 * _K)
        if j % 2 == 0
    ]
    for r in range(half, _R):
        for c in range(_VPR):
            st_v[r, pl.ds(c * _LANES, _LANES)] = regs[c]
    second = [
        pltpu.async_copy(
            st_v.at[pl.ds(half, half)],
            out_hbm.at[pl.ds(base + j * half, half)],
            osem,
        )
        for j in range(2 * _K)
        if j % 2 == 1
    ]
    for c in first + second:
        c.wait()


def kernel(idx, shape, table):
    if idx is None:
        idx = _NUM_CLASSES
    row = jnp.asarray(idx, jnp.int32) + (
        jnp.asarray(shape[0], jnp.int32) - jnp.int32(_BATCH)
    )
    idx_arr = jnp.full((1,), row, dtype=jnp.int32)
    return _bcast_row(idx_arr, table)


# speedup vs baseline: 1.0392x; 1.0392x over previous
import functools

import jax
import jax.numpy as jnp
from jax import lax
from jax.experimental import pallas as pl
from jax.experimental.pallas import tpu as pltpu
from jax.experimental.pallas import tpu_sc as plsc

_NUM_CLASSES = 1000
_EMBED = 128
_BATCH = 16384

_NC = 2
_NS = 16
_NW = _NC * _NS
_BPW = _BATCH // _NW
_R = 128
_K = _BPW // _R
_LANES = 16
_VPR = _EMBED // _LANES


@functools.partial(
    pl.kernel,
    out_type=jax.ShapeDtypeStruct((_BATCH, _EMBED), jnp.float32),
    mesh=plsc.VectorSubcoreMesh(core_axis_name="c", subcore_axis_name="s"),
    scratch_types=[
        pltpu.VMEM((1,), jnp.int32),
        pltpu.VMEM((_R, _EMBED), jnp.float32),
        pltpu.SemaphoreType.DMA,
        pltpu.SemaphoreType.DMA,
    ],
)
def _bcast_row(idx_hbm, table_hbm, out_hbm, idx_v, st_v, gsem, osem):
    wid = lax.axis_index("s") * _NC + lax.axis_index("c")
    base = wid * _BPW
    pltpu.sync_copy(idx_hbm, idx_v)
    pltpu.async_copy(table_hbm.at[idx_v], st_v.at[pl.ds(0, 1)], gsem).wait()
    regs = [st_v[0, pl.ds(c * _LANES, _LANES)] for c in range(_VPR)]
    for r in range(1, _R):
        for c in range(_VPR):
            st_v[r, pl.ds(c * _LANES, _LANES)] = regs[c]
    copies = [
        pltpu.async_copy(st_v, out_hbm.at[pl.ds(base + j * _R, _R)], osem)
        for j in range(_K)
    ]
    for c in copies:
        c.wait()


def kernel(idx, shape, table):
    if idx is None:
        idx = _NUM_CLASSES
    row = jnp.asarray(idx, jnp.int32) + (
        jnp.asarray(shape[0], jnp.int32) - jnp.int32(_BATCH)
    )
    idx_arr = jnp.full((1,), row, dtype=jnp.int32)
    return _bcast_row(idx_arr, table)


# speedup vs baseline: 1.0906x; 1.0494x over previous
import functools

import jax
import jax.numpy as jnp
from jax import lax
from jax.experimental import pallas as pl
from jax.experimental.pallas import tpu as pltpu
from jax.experimental.pallas import tpu_sc as plsc

_NUM_CLASSES = 1000
_EMBED = 128
_BATCH = 16384

_NC = 2
_NS = 16
_NW = _NC * _NS
_BPW = _BATCH // _NW
_R = 128
_K = _BPW // _R
_LANES = 16
_VPR = _EMBED // _LANES


@functools.partial(
    pl.kernel,
    out_type=jax.ShapeDtypeStruct((_BATCH, _EMBED), jnp.float32),
    mesh=plsc.VectorSubcoreMesh(core_axis_name="c", subcore_axis_name="s"),
    scratch_types=[
        pltpu.VMEM((1,), jnp.int32),
        pltpu.VMEM((1, _EMBED), jnp.float32),
        pltpu.VMEM((_R, _EMBED), jnp.float32),
        pltpu.SemaphoreType.DMA,
        pltpu.SemaphoreType.DMA,
    ],
)
def _bcast_row(idx_hbm, table_hbm, out_hbm, idx_v, row_v, st_v, gsem, osem):
    wid = lax.axis_index("s") * _NC + lax.axis_index("c")
    base = wid * _BPW
    pltpu.sync_copy(idx_hbm, idx_v)
    pltpu.async_copy(table_hbm.at[idx_v], row_v, gsem).wait()
    regs = [row_v[0, pl.ds(c * _LANES, _LANES)] for c in range(_VPR)]
    for r in range(_R):
        for c in range(_VPR):
            st_v[r, pl.ds(c * _LANES, _LANES)] = regs[c]
    copies = [
        pltpu.async_copy(st_v, out_hbm.at[pl.ds(base + j * _R, _R)], osem)
        for j in range(_K)
    ]
    for c in copies:
        c.wait()


def kernel(idx, shape, table):
    if idx is None:
        idx = _NUM_CLASSES
    row = jnp.asarray(idx, jnp.int32) + (
        jnp.asarray(shape[0], jnp.int32) - jnp.int32(_BATCH)
    )
    idx_arr = jnp.full((1,), row, dtype=jnp.int32)
    return _bcast_row(idx_arr, table)
